# baseline (speedup 1.0000x reference)
import jax
import jax.numpy as jnp
from jax import lax
from jax.experimental import pallas as pl
from jax.experimental.pallas import tpu as pltpu
from jax.experimental.pallas import tpu_sc as plsc

N = 50000
E = 800000
G = 64
D_IN = 336
D_H = 100
D_P = 128
D_OUT = 29

NC = 2
NS = 16
LANES = 16

NB = 8
BUCKET = 6272
DUMP = BUCKET
ACC_ROWS = BUCKET + 8

SEGS = 128
SEG_E = 6272
E_PAD = SEGS * SEG_E
SEG_ITERS = SEG_E // LANES
CAP = 6656
CHUNK = 512
ROWS_PER_TILE = BUCKET // NS
LAST_ROWS = N - (NB - 1) * BUCKET - (NS - 1) * ROWS_PER_TILE


def _iota16():
    return lax.iota(jnp.int32, LANES)


def _mesh():
    return plsc.VectorSubcoreMesh(core_axis_name="c", subcore_axis_name="s",
                                  num_cores=NC, num_subcores=NS)


def _bin_body(src2, dst2, bufs_src, bufs_ldst, counts,
              src_in, dst_in,
              sbuf0, sbuf1, sbuf2, sbuf3, sbuf4, sbuf5, sbuf6, sbuf7,
              lbuf0, lbuf1, lbuf2, lbuf3, lbuf4, lbuf5, lbuf6, lbuf7,
              cvec, sem):
    sbuf = (sbuf0, sbuf1, sbuf2, sbuf3, sbuf4, sbuf5, sbuf6, sbuf7)
    lbuf = (lbuf0, lbuf1, lbuf2, lbuf3, lbuf4, lbuf5, lbuf6, lbuf7)
    c = lax.axis_index("c")
    s = lax.axis_index("s")
    w = c * NS + s
    for j in range(SEGS // (NC * NS)):
        seg = w * (SEGS // (NC * NS)) + j
        pltpu.sync_copy(src2.at[seg], src_in)
        pltpu.sync_copy(dst2.at[seg], dst_in)

        def body(i, offs):
            sl = pl.ds(i * LANES, LANES)
            sv = src_in[sl]
            dv = dst_in[sl]
            new_offs = []
            for b in range(NB):
                lo = b * BUCKET
                hi = (b + 1) * BUCKET
                m = dv >= lo if b > 0 else dv >= 0
                if b < NB - 1:
                    m = jnp.logical_and(m, dv < hi)
                off = offs[b]
                mi = m.astype(jnp.int32)
                pos = off + plsc.cumsum(mi) - 1
                plsc.store_scatter(sbuf[b], [pos], sv, mask=m)
                plsc.store_scatter(lbuf[b], [pos], dv - lo, mask=m)
                new_offs.append(off + jnp.sum(mi))
            return tuple(new_offs)

        offs = lax.fori_loop(0, SEG_ITERS, body,
                             tuple(jnp.int32(0) for _ in range(NB)))
        cv = jnp.zeros((LANES,), jnp.int32)
        for b in range(NB):
            cv = jnp.where(_iota16() == b, offs[b], cv)
        cvec[pl.ds(0, LANES)] = cv
        pltpu.sync_copy(cvec, counts.at[seg])
        for b in range(NB):
            pltpu.sync_copy(sbuf[b], bufs_src.at[seg, b])
            pltpu.sync_copy(lbuf[b], bufs_ldst.at[seg, b])


def _bin_edges(src2, dst2):
    return pl.kernel(
        _bin_body,
        out_type=(
            jax.ShapeDtypeStruct((SEGS, NB, CAP), jnp.int32),
            jax.ShapeDtypeStruct((SEGS, NB, CAP), jnp.int32),
            jax.ShapeDtypeStruct((SEGS, 128), jnp.int32),
        ),
        mesh=_mesh(),
        compiler_params=pltpu.CompilerParams(needs_layout_passes=False),
        scratch_types=[
            pltpu.VMEM((SEG_E,), jnp.int32),
            pltpu.VMEM((SEG_E,), jnp.int32),
        ] + [pltpu.VMEM((CAP,), jnp.int32) for _ in range(2 * NB)] + [
            pltpu.VMEM((128,), jnp.int32),
            pltpu.SemaphoreType.DMA,
        ],
    )(src2, dst2)


def _seg_body(p_hbm, bufs_src, bufs_ldst, counts, zeros_hbm, out_hbm,
              src_idx, ldst_idx, rows, zbuf, cvec, zidx, acc, sem):
    c = lax.axis_index("c")
    s = lax.axis_index("s")

    pltpu.sync_copy(zeros_hbm, zbuf)

    def phase(j, _):
        b = c * (NB // NC) + j
        base = s * ROWS_PER_TILE

        def zchunk(u, _):
            for v in range(2):
                zidx[pl.ds(v * LANES, LANES)] = (base + u * 32 + v * LANES
                                                 + _iota16())
            pltpu.sync_copy(zbuf, acc.at[zidx])
            return 0

        lax.fori_loop(0, 12, zchunk, 0)
        zidx[pl.ds(0, LANES)] = base + 384 + _iota16()
        pltpu.sync_copy(zbuf.at[pl.ds(0, 16)], acc.at[zidx.at[pl.ds(0, 16)]])
        plsc.subcore_barrier()

        def segloop(k, _):
            seg = s + k * NS
            pltpu.sync_copy(counts.at[seg], cvec)
            cnt = jnp.sum(jnp.where(_iota16() == b, cvec[pl.ds(0, LANES)], 0))
            nch = (cnt + (CHUNK - 1)) // CHUNK

            def chunk_body(i, _):
                pltpu.sync_copy(bufs_src.at[seg, b, pl.ds(i * CHUNK, CHUNK)],
                                src_idx)
                pltpu.sync_copy(bufs_ldst.at[seg, b, pl.ds(i * CHUNK, CHUNK)],
                                ldst_idx)

                @pl.when((i + 1) * CHUNK > cnt)
                def _fix_tail():
                    for u in range(CHUNK // LANES):
                        sl = pl.ds(u * LANES, LANES)
                        pos = i * CHUNK + u * LANES + _iota16()
                        m = pos < cnt
                        src_idx[sl] = jnp.where(m, src_idx[sl], 0)
                        ldst_idx[sl] = jnp.where(m, ldst_idx[sl], DUMP)

                pltpu.async_copy(p_hbm.at[src_idx], rows, sem).wait()
                pltpu.sync_copy(rows, acc.at[ldst_idx], add=True)
                return 0

            lax.fori_loop(0, nch, chunk_body, 0)
            return 0

        lax.fori_loop(0, SEGS // NS, segloop, 0)
        plsc.subcore_barrier()

        start = b * BUCKET + s * ROWS_PER_TILE

        @pl.when(jnp.logical_or(b < NB - 1, s < NS - 1))
        def _wb_full():
            pltpu.sync_copy(acc.at[pl.ds(s * ROWS_PER_TILE, ROWS_PER_TILE)],
                            out_hbm.at[pl.ds(start, ROWS_PER_TILE)])

        @pl.when(jnp.logical_and(b == NB - 1, s == NS - 1))
        def _wb_last():
            pltpu.sync_copy(acc.at[pl.ds(s * ROWS_PER_TILE, LAST_ROWS)],
                            out_hbm.at[pl.ds(start, LAST_ROWS)])

        plsc.subcore_barrier()
        return 0

    lax.fori_loop(0, NB // NC, phase, 0)


def _segment_sum(p, bufs_src, bufs_ldst, counts, zeros):
    return pl.kernel(
        _seg_body,
        out_type=jax.ShapeDtypeStruct((N, D_P), jnp.float32),
        mesh=_mesh(),
        compiler_params=pltpu.CompilerParams(needs_layout_passes=False),
        scratch_types=[
            pltpu.VMEM((CHUNK,), jnp.int32),
            pltpu.VMEM((CHUNK,), jnp.int32),
            pltpu.VMEM((CHUNK, D_P), jnp.float32),
            pltpu.VMEM((32, D_P), jnp.float32),
            pltpu.VMEM((128,), jnp.int32),
            pltpu.VMEM((32,), jnp.int32),
            pltpu.VMEM_SHARED((ACC_ROWS, D_P), jnp.float32),
            pltpu.SemaphoreType.DMA,
        ],
    )(p, bufs_src, bufs_ldst, counts, zeros)


_TC_ROWS = 1000
_GRID = N // _TC_ROWS


def _dotT(a, w):
    return lax.dot_general(a, w, (((1,), (1,)), ((), ())),
                           preferred_element_type=jnp.float32)


def _proj1_kernel(x_ref, wr_ref, wo_ref, b_ref, p_ref, r_ref):
    xb = x_ref[...]
    p_ref[...] = _dotT(xb, wr_ref[...])
    r_ref[...] = _dotT(xb, wo_ref[...]) + b_ref[...]


def _proj1(x, wr, wo, b):
    return pl.pallas_call(
        _proj1_kernel,
        grid=(_GRID,),
        in_specs=[
            pl.BlockSpec((_TC_ROWS, D_IN), lambda i: (i, 0)),
            pl.BlockSpec((D_P, D_IN), lambda i: (0, 0)),
            pl.BlockSpec((D_P, D_IN), lambda i: (0, 0)),
            pl.BlockSpec((1, D_P), lambda i: (0, 0)),
        ],
        out_specs=[
            pl.BlockSpec((_TC_ROWS, D_P), lambda i: (i, 0)),
            pl.BlockSpec((_TC_ROWS, D_P), lambda i: (i, 0)),
        ],
        out_shape=[
            jax.ShapeDtypeStruct((N, D_P), jnp.float32),
            jax.ShapeDtypeStruct((N, D_P), jnp.float32),
        ],
    )(x, wr, wo, b)


def _epiproj_kernel(agg_ref, rp_ref, wr_ref, wo_ref, b_ref, p_ref, r_ref):
    h = jnp.maximum(agg_ref[...] + rp_ref[...], 0.0)
    p_ref[...] = _dotT(h, wr_ref[...])
    r_ref[...] = _dotT(h, wo_ref[...]) + b_ref[...]


def _epiproj(agg, rprev, wr, wo, b):
    return pl.pallas_call(
        _epiproj_kernel,
        grid=(_GRID,),
        in_specs=[
            pl.BlockSpec((_TC_ROWS, D_P), lambda i: (i, 0)),
            pl.BlockSpec((_TC_ROWS, D_P), lambda i: (i, 0)),
            pl.BlockSpec((D_P, D_P), lambda i: (0, 0)),
            pl.BlockSpec((D_P, D_P), lambda i: (0, 0)),
            pl.BlockSpec((1, D_P), lambda i: (0, 0)),
        ],
        out_specs=[
            pl.BlockSpec((_TC_ROWS, D_P), lambda i: (i, 0)),
            pl.BlockSpec((_TC_ROWS, D_P), lambda i: (i, 0)),
        ],
        out_shape=[
            jax.ShapeDtypeStruct((N, D_P), jnp.float32),
            jax.ShapeDtypeStruct((N, D_P), jnp.float32),
        ],
    )(agg, rprev, wr, wo, b)


def _final_kernel(agg_ref, rp_ref, batch_ref, w1_ref, b1_ref, w2_ref, b2_ref,
                  w4_ref, b4_ref, pooled_ref, out_ref):
    pi = pl.program_id(0)

    @pl.when(pi == 0)
    def _init():
        pooled_ref[...] = jnp.zeros_like(pooled_ref)

    h = jnp.maximum(agg_ref[...] + rp_ref[...], 0.0)
    hb = jnp.concatenate(
        [h[:, :D_H], jnp.ones((_TC_ROWS, 1), jnp.float32),
         jnp.zeros((_TC_ROWS, 27), jnp.float32)], axis=1)
    bt = batch_ref[...].reshape(_TC_ROWS, 1)
    oh = (bt == lax.broadcasted_iota(jnp.int32, (_TC_ROWS, G), 1)
          ).astype(jnp.float32)
    pooled_ref[...] += lax.dot_general(oh, hb, (((0,), (0,)), ((), ())),
                                       preferred_element_type=jnp.float32)

    @pl.when(pi == _GRID - 1)
    def _head():
        acc = pooled_ref[...]
        sums = acc[:, :D_H]
        cnt = acc[:, D_H:D_H + 1]
        mean = sums / jnp.maximum(cnt, 1.0)
        h1 = jnp.maximum(_dotT(mean, w1_ref[...]) + b1_ref[...], 0.0)
        h2 = jnp.maximum(_dotT(h1, w2_ref[...]) + b2_ref[...], 0.0)
        out_ref[...] = _dotT(h2, w4_ref[...]) + b4_ref[...]


def _final(agg, rprev, batch3, w1, b1, w2, b2, w4, b4):
    _, out = pl.pallas_call(
        _final_kernel,
        grid=(_GRID,),
        in_specs=[
            pl.BlockSpec((_TC_ROWS, D_P), lambda i: (i, 0)),
            pl.BlockSpec((_TC_ROWS, D_P), lambda i: (i, 0)),
            pl.BlockSpec((1, 1, _TC_ROWS), lambda i: (i, 0, 0)),
            pl.BlockSpec((D_H, D_H), lambda i: (0, 0)),
            pl.BlockSpec((1, D_H), lambda i: (0, 0)),
            pl.BlockSpec((D_H, D_H), lambda i: (0, 0)),
            pl.BlockSpec((1, D_H), lambda i: (0, 0)),
            pl.BlockSpec((D_OUT, D_H), lambda i: (0, 0)),
            pl.BlockSpec((1, D_OUT), lambda i: (0, 0)),
        ],
        out_specs=[
            pl.BlockSpec((G, 128), lambda i: (0, 0)),
            pl.BlockSpec((G, D_OUT), lambda i: (0, 0)),
        ],
        out_shape=[
            jax.ShapeDtypeStruct((G, 128), jnp.float32),
            jax.ShapeDtypeStruct((G, D_OUT), jnp.float32),
        ],
    )(agg, rprev, batch3, w1, b1, w2, b2, w4, b4)
    return out


def _pad_w(w):
    return jnp.pad(w, ((0, D_P - D_H), (0, 0)))


def _pad_w_sq(w):
    return jnp.pad(w, ((0, D_P - D_H), (0, D_P - D_H)))


def _pad_b(b):
    return jnp.pad(b, (0, D_P - D_H)).reshape(1, D_P)


def kernel(x, edge_index, batch,
           W_rel1, b1, W_root1,
           W_rel2, b2, W_root2,
           W_rel3, b3, W_root3,
           W_rel4, b4, W_root4,
           W_rel5, b5, W_root5,
           lin1_W, lin1_b, lin2_W, lin2_b, lin4_W, lin4_b):
    ei = edge_index.astype(jnp.int32)
    pad = E_PAD - E
    src2 = jnp.concatenate([ei[0], jnp.zeros((pad,), jnp.int32)]
                           ).reshape(SEGS, SEG_E)
    dst2 = jnp.concatenate([ei[1], jnp.full((pad,), -1, jnp.int32)]
                           ).reshape(SEGS, SEG_E)
    batch3 = batch.astype(jnp.int32).reshape(_GRID, 1, _TC_ROWS)

    bufs_src, bufs_ldst, counts = _bin_edges(src2, dst2)
    zeros = jnp.zeros((32, D_P), jnp.float32)

    p, r = _proj1(x, _pad_w(W_rel1), _pad_w(W_root1), _pad_b(b1))
    for wr, wo, bb in ((W_rel2, W_root2, b2), (W_rel3, W_root3, b3),
                       (W_rel4, W_root4, b4), (W_rel5, W_root5, b5)):
        agg = _segment_sum(p, bufs_src, bufs_ldst, counts, zeros)
        p, r = _epiproj(agg, r, _pad_w_sq(wr), _pad_w_sq(wo), _pad_b(bb))
    agg = _segment_sum(p, bufs_src, bufs_ldst, counts, zeros)
    return _final(agg, r, batch3,
                  lin1_W, lin1_b.reshape(1, D_H),
                  lin2_W, lin2_b.reshape(1, D_H),
                  lin4_W, lin4_b.reshape(1, D_OUT))
